# baseline (speedup 1.0000x reference)
import functools

import jax
import jax.numpy as jnp
from jax import lax
from jax.experimental import pallas as pl
from jax.experimental.pallas import tpu as pltpu
from jax.experimental.pallas import tpu_sc as plsc

_N = 10000
_D = 128
_E = 320000
_BLK = 1000

_NCORES = 2
_NSUB = 16
_NW = _NCORES * _NSUB
_CH = 128
_NP = 10240
_RPT = _NP // _NSUB
_NCH0 = 79
_NCH1 = 79
_NCHMAX = 79
_EPT = _NCHMAX * _CH
_EPW0 = 10000
_EPW1 = 10000
_EPAD = _NW * _EPT


def _sc_edge_body(x_hbm, src_hbm, dst_hbm, att_hbm, u_out, s_out,
                  idxs_v, idxd_v, att_v, rows_v, zrow_v, zs_v, u_sh, s_sh,
                  sem):
    c = lax.axis_index("c")
    s = lax.axis_index("s")
    w = c * _NSUB + s
    zv = jnp.zeros((16,), jnp.float32)

    def zrow_body(i, carry):
        for j in range(8):
            zrow_v[i, pl.ds(j * 16, 16)] = zv
        return carry
    lax.fori_loop(0, _CH, zrow_body, 0)

    def zs_body(i, carry):
        zs_v[pl.ds(i * 16, 16)] = zv
        return carry
    lax.fori_loop(0, _RPT // 16, zs_body, 0)

    row0 = s * _RPT
    for t in range(_RPT // _CH):
        pltpu.sync_copy(zrow_v, u_sh.at[pl.ds(row0 + t * _CH, _CH)])
    pltpu.sync_copy(zs_v, s_sh.at[pl.ds(row0, _RPT)])
    plsc.subcore_barrier()

    base = w * _EPT
    nch = _NCH0 + (_NCH1 - _NCH0) * c

    def chunk_body(i, carry):
        @pl.when(i < nch)
        def _():
            off = base + i * _CH
            pltpu.sync_copy(src_hbm.at[pl.ds(off, _CH)], idxs_v)
            pltpu.sync_copy(dst_hbm.at[pl.ds(off, _CH)], idxd_v)
            pltpu.sync_copy(att_hbm.at[pl.ds(off, _CH)], att_v)
            pltpu.async_copy(x_hbm.at[idxs_v], rows_v, sem).wait()

            def scale_body(g, carry2):
                av = att_v[pl.ds(g * 16, 16)]
                for l in range(16):
                    a = av[l]
                    k = g * 16 + l
                    for j in range(8):
                        sl = pl.ds(j * 16, 16)
                        rows_v[k, sl] = rows_v[k, sl] * a
                return carry2
            lax.fori_loop(0, _CH // 16, scale_body, 0)

            pltpu.sync_copy(rows_v, u_sh.at[idxd_v], add=True)
            pltpu.sync_copy(att_v, s_sh.at[idxd_v], add=True)
        return carry
    lax.fori_loop(0, _NCHMAX, chunk_body, 0)
    plsc.subcore_barrier()

    pltpu.sync_copy(u_sh.at[pl.ds(row0, _RPT)],
                    u_out.at[c, pl.ds(row0, _RPT)])
    pltpu.sync_copy(s_sh.at[pl.ds(row0, _RPT)],
                    s_out.at[c, pl.ds(row0, _RPT)])


def _sc_edge_pass(x, src, dst, att):
    mesh = plsc.VectorSubcoreMesh(core_axis_name="c", subcore_axis_name="s")
    fn = functools.partial(
        pl.kernel,
        mesh=mesh,
        out_type=[
            jax.ShapeDtypeStruct((_NCORES, _NP, _D), jnp.float32),
            jax.ShapeDtypeStruct((_NCORES, _NP), jnp.float32),
        ],
        scratch_types=[
            pltpu.VMEM((_CH,), jnp.int32),
            pltpu.VMEM((_CH,), jnp.int32),
            pltpu.VMEM((_CH,), jnp.float32),
            pltpu.VMEM((_CH, _D), jnp.float32),
            pltpu.VMEM((_CH, _D), jnp.float32),
            pltpu.VMEM((_RPT,), jnp.float32),
            pltpu.VMEM_SHARED((_NP, _D), jnp.float32),
            pltpu.VMEM_SHARED((_NP,), jnp.float32),
            pltpu.SemaphoreType.DMA,
        ],
    )(_sc_edge_body)
    return fn(x, src, dst, att)


def _dense_body(num0_ref, num1_ref, den0_ref, den1_ref, x_ref, w_ref,
                b_ref, g_ref, be_ref, o_ref):
    num = num0_ref[...] + num1_ref[...]
    den = den0_ref[...] + den1_ref[...]
    x = x_ref[...]
    aggr = jnp.where(den > 0.0, num / jnp.where(den > 0.0, den, 1.0), 0.0)
    gelu = 0.5 * aggr * (1.0 + jax.lax.erf(aggr * 0.7071067811865476))
    h = gelu + x
    t = jax.lax.dot_general(h, w_ref[...], (((1,), (1,)), ((), ())),
                            preferred_element_type=jnp.float32)
    t = t + b_ref[...]
    mu = jnp.mean(t, axis=-1, keepdims=True)
    var = jnp.mean((t - mu) ** 2, axis=-1, keepdims=True)
    o_ref[...] = (t - mu) * jax.lax.rsqrt(var + 1e-5) * g_ref[...] + be_ref[...]


def _dense_layer(num0, num1, den0, den1, x, w, b, g, be):
    row_spec = pl.BlockSpec((_BLK, _D), lambda i: (i, 0))
    den_spec = pl.BlockSpec((_BLK, 1), lambda i: (i, 0))
    vec_spec = pl.BlockSpec((1, _D), lambda i: (0, 0))
    return pl.pallas_call(
        _dense_body,
        grid=(_N // _BLK,),
        in_specs=[row_spec, row_spec, den_spec, den_spec, row_spec,
                  pl.BlockSpec((_D, _D), lambda i: (0, 0)),
                  vec_spec, vec_spec, vec_spec],
        out_specs=row_spec,
        out_shape=jax.ShapeDtypeStruct((_N, _D), jnp.float32),
    )(num0, num1, den0, den1, x, w, b, g, be)


_LENS = [_EPW0] * _NSUB + [_EPW1] * _NSUB
_STARTS = [sum(_LENS[:w]) for w in range(_NW)]


def _split_regions(arr, dtype):
    ap = jnp.concatenate([arr, jnp.zeros((_EPT,), dtype)])
    regions = []
    for w in range(_NW):
        r = ap[_STARTS[w]:_STARTS[w] + _EPT]
        if _LENS[w] < _EPT:
            r = jnp.where(jnp.arange(_EPT) < _LENS[w], r,
                          jnp.zeros((), dtype))
        regions.append(r)
    return jnp.concatenate(regions)


def kernel(node_attr, edge_index, batch_idx, adv_atts, W0, b0, g0, be0,
           W1, b1, g1, be1):
    src = _split_regions(edge_index[0], jnp.int32)
    dst = _split_regions(edge_index[1], jnp.int32)
    att0 = _split_regions(adv_atts[0], jnp.float32)
    att1 = _split_regions(adv_atts[1], jnp.float32)

    x = node_attr
    for att, w, b, g, be in ((att0, W0, b0, g0, be0),
                             (att1, W1, b1, g1, be1)):
        u, sden = _sc_edge_pass(x, src, dst, att)
        x = _dense_layer(u[0, :_N], u[1, :_N],
                         sden[0, :_N].reshape(_N, 1),
                         sden[1, :_N].reshape(_N, 1),
                         x, w, b.reshape(1, _D), g.reshape(1, _D),
                         be.reshape(1, _D))
    return x


# speedup vs baseline: 1.1563x; 1.1563x over previous
import functools

import jax
import jax.numpy as jnp
from jax import lax
from jax.experimental import pallas as pl
from jax.experimental.pallas import tpu as pltpu
from jax.experimental.pallas import tpu_sc as plsc

_N = 10000
_D = 128
_E = 320000
_BLK = 1000

_NCORES = 2
_NSUB = 16
_NW = _NCORES * _NSUB
_CH = 128
_NP = 10240
_RPT = _NP // _NSUB
_NCH0 = 76
_NCH1 = 81
_NCHMAX = 81
_EPT = _NCHMAX * _CH
_EPW0 = 9728
_EPW1 = 10272
_EPAD = _NW * _EPT


def _sc_edge_body(x_hbm, src_hbm, dst_hbm, att_hbm, u_out, s_out,
                  idxs_v, idxd_v, att_v, rows_v, zrow_v, zs_v, u_sh, s_sh,
                  sem):
    c = lax.axis_index("c")
    s = lax.axis_index("s")
    w = c * _NSUB + s
    zv = jnp.zeros((16,), jnp.float32)

    def zrow_body(i, carry):
        for j in range(8):
            zrow_v[i, pl.ds(j * 16, 16)] = zv
        return carry
    lax.fori_loop(0, _CH, zrow_body, 0)

    def zs_body(i, carry):
        zs_v[pl.ds(i * 16, 16)] = zv
        return carry
    lax.fori_loop(0, _RPT // 16, zs_body, 0)

    row0 = s * _RPT
    for t in range(_RPT // _CH):
        pltpu.sync_copy(zrow_v, u_sh.at[pl.ds(row0 + t * _CH, _CH)])
    pltpu.sync_copy(zs_v, s_sh.at[pl.ds(row0, _RPT)])
    plsc.subcore_barrier()

    base = w * _EPT
    nch = _NCH0 + (_NCH1 - _NCH0) * c

    def chunk_body(i, carry):
        @pl.when(i < nch)
        def _():
            off = base + i * _CH
            pltpu.sync_copy(src_hbm.at[pl.ds(off, _CH)], idxs_v)
            pltpu.sync_copy(dst_hbm.at[pl.ds(off, _CH)], idxd_v)
            pltpu.sync_copy(att_hbm.at[pl.ds(off, _CH)], att_v)
            pltpu.async_copy(x_hbm.at[idxs_v], rows_v, sem).wait()

            def scale_body(g, carry2):
                av = att_v[pl.ds(g * 16, 16)]
                for l in range(16):
                    a = av[l]
                    k = g * 16 + l
                    for j in range(8):
                        sl = pl.ds(j * 16, 16)
                        rows_v[k, sl] = rows_v[k, sl] * a
                return carry2
            lax.fori_loop(0, _CH // 16, scale_body, 0)

            pltpu.sync_copy(rows_v, u_sh.at[idxd_v], add=True)
            pltpu.sync_copy(att_v, s_sh.at[idxd_v], add=True)
        return carry
    lax.fori_loop(0, _NCHMAX, chunk_body, 0)
    plsc.subcore_barrier()

    pltpu.sync_copy(u_sh.at[pl.ds(row0, _RPT)],
                    u_out.at[c, pl.ds(row0, _RPT)])
    pltpu.sync_copy(s_sh.at[pl.ds(row0, _RPT)],
                    s_out.at[c, pl.ds(row0, _RPT)])


def _sc_edge_pass(x, src, dst, att):
    mesh = plsc.VectorSubcoreMesh(core_axis_name="c", subcore_axis_name="s")
    fn = functools.partial(
        pl.kernel,
        mesh=mesh,
        out_type=[
            jax.ShapeDtypeStruct((_NCORES, _NP, _D), jnp.float32),
            jax.ShapeDtypeStruct((_NCORES, _NP), jnp.float32),
        ],
        scratch_types=[
            pltpu.VMEM((_CH,), jnp.int32),
            pltpu.VMEM((_CH,), jnp.int32),
            pltpu.VMEM((_CH,), jnp.float32),
            pltpu.VMEM((_CH, _D), jnp.float32),
            pltpu.VMEM((_CH, _D), jnp.float32),
            pltpu.VMEM((_RPT,), jnp.float32),
            pltpu.VMEM_SHARED((_NP, _D), jnp.float32),
            pltpu.VMEM_SHARED((_NP,), jnp.float32),
            pltpu.SemaphoreType.DMA,
        ],
    )(_sc_edge_body)
    return fn(x, src, dst, att)


def _dense_body(num0_ref, num1_ref, den0_ref, den1_ref, x_ref, w_ref,
                b_ref, g_ref, be_ref, o_ref):
    num = num0_ref[...] + num1_ref[...]
    den = den0_ref[...] + den1_ref[...]
    x = x_ref[...]
    aggr = jnp.where(den > 0.0, num / jnp.where(den > 0.0, den, 1.0), 0.0)
    gelu = 0.5 * aggr * (1.0 + jax.lax.erf(aggr * 0.7071067811865476))
    h = gelu + x
    t = jax.lax.dot_general(h, w_ref[...], (((1,), (1,)), ((), ())),
                            preferred_element_type=jnp.float32)
    t = t + b_ref[...]
    mu = jnp.mean(t, axis=-1, keepdims=True)
    var = jnp.mean((t - mu) ** 2, axis=-1, keepdims=True)
    o_ref[...] = (t - mu) * jax.lax.rsqrt(var + 1e-5) * g_ref[...] + be_ref[...]


def _dense_layer(num0, num1, den0, den1, x, w, b, g, be):
    row_spec = pl.BlockSpec((_BLK, _D), lambda i: (i, 0))
    den_spec = pl.BlockSpec((_BLK, 1), lambda i: (i, 0))
    vec_spec = pl.BlockSpec((1, _D), lambda i: (0, 0))
    return pl.pallas_call(
        _dense_body,
        grid=(_N // _BLK,),
        in_specs=[row_spec, row_spec, den_spec, den_spec, row_spec,
                  pl.BlockSpec((_D, _D), lambda i: (0, 0)),
                  vec_spec, vec_spec, vec_spec],
        out_specs=row_spec,
        out_shape=jax.ShapeDtypeStruct((_N, _D), jnp.float32),
    )(num0, num1, den0, den1, x, w, b, g, be)


_LENS = [_EPW0] * _NSUB + [_EPW1] * _NSUB
_STARTS = [sum(_LENS[:w]) for w in range(_NW)]


def _split_regions(arr, dtype):
    ap = jnp.concatenate([arr, jnp.zeros((_EPT,), dtype)])
    regions = []
    for w in range(_NW):
        r = ap[_STARTS[w]:_STARTS[w] + _EPT]
        if _LENS[w] < _EPT:
            r = jnp.where(jnp.arange(_EPT) < _LENS[w], r,
                          jnp.zeros((), dtype))
        regions.append(r)
    return jnp.concatenate(regions)


def kernel(node_attr, edge_index, batch_idx, adv_atts, W0, b0, g0, be0,
           W1, b1, g1, be1):
    src = _split_regions(edge_index[0], jnp.int32)
    dst = _split_regions(edge_index[1], jnp.int32)
    att0 = _split_regions(adv_atts[0], jnp.float32)
    att1 = _split_regions(adv_atts[1], jnp.float32)

    x = node_attr
    for att, w, b, g, be in ((att0, W0, b0, g0, be0),
                             (att1, W1, b1, g1, be1)):
        u, sden = _sc_edge_pass(x, src, dst, att)
        x = _dense_layer(u[0, :_N], u[1, :_N],
                         sden[0, :_N].reshape(_N, 1),
                         sden[1, :_N].reshape(_N, 1),
                         x, w, b.reshape(1, _D), g.reshape(1, _D),
                         be.reshape(1, _D))
    return x
